# baseline (speedup 1.0000x reference)
import functools

import numpy as np

import jax
import jax.numpy as jnp
from jax import lax
from jax.experimental import pallas as pl
from jax.experimental.pallas import tpu as pltpu
from jax.experimental.pallas import tpu_sc as plsc

POOL = 256
CH = 3
SZ = 32
EMBED = 768
B = 128
L = 196
PATCH = 16
IMG = 224
G = IMG // PATCH
WIN = 4
K = CH * WIN * WIN
EPS = 1e-12
RB = 8
NEB = L // RB
NB = 2 * L // RB


def _resize_mat():
    x = np.arange(IMG)
    s = (x + 0.5) * (SZ / IMG) - 0.5
    lo = np.floor(s).astype(int)
    w = s - lo
    r = np.zeros((IMG, SZ), np.float64)
    for i in range(IMG):
        for j, wt in ((lo[i], 1.0 - w[i]), (lo[i] + 1, w[i])):
            r[i, min(max(j, 0), SZ - 1)] += wt
    return r


_RY = _resize_mat()
_P0 = np.zeros(G, np.int32)
_RS = np.zeros((G, WIN, PATCH), np.float32)
for _g in range(G):
    _rows = _RY[PATCH * _g : PATCH * (_g + 1)]
    _nz = np.nonzero(np.abs(_rows).sum(0))[0]
    _p0 = min(int(_nz.min()), SZ - WIN)
    _P0[_g] = _p0
    _RS[_g] = _rows[:, _p0 : _p0 + WIN].T.astype(np.float32)

_Y0 = _P0.copy()
_X0 = [int(v) for v in _P0]


_MB = 8


def _mean_body(x_ref, o_ref, of_ref):
    j2 = pl.program_id(1)

    @pl.when(j2 == 0)
    def _():
        o_ref[...] = jnp.mean(x_ref[...], axis=1)

    a = x_ref[:, pl.ds(pl.multiple_of(8 * j2, 8), RB), :]
    bch = x_ref[:, pl.ds(pl.multiple_of(8 * j2 + 8, 8), RB), :]
    of_ref[:, 0] = jnp.concatenate([a[:, WIN:, :], bch[:, :WIN, :]], axis=1)


def _mean_call(x_embed):
    return pl.pallas_call(
        _mean_body,
        grid=(B // _MB, NEB),
        in_specs=[pl.BlockSpec((_MB, L, EMBED), lambda i, j2: (i, 0, 0))],
        out_specs=[
            pl.BlockSpec((_MB, EMBED), lambda i, j2: (i, 0)),
            pl.BlockSpec((_MB, 1, RB, EMBED),
                         lambda i, j2: (i, j2 + NEB + 1, 0, 0)),
        ],
        out_shape=[jax.ShapeDtypeStruct((B, EMBED), jnp.float32),
                   jax.ShapeDtypeStruct((B, NB, RB, EMBED), jnp.float32)],
    )(x_embed)


def _sim_body(xm_ref, pk_ref, sim_ref, idx_ref, rs_ref):
    xm = xm_ref[...]
    pk = pk_ref[...]
    pk_n = pk * lax.rsqrt(jnp.maximum(jnp.sum(pk * pk, axis=1, keepdims=True), EPS))
    xm_n = xm * lax.rsqrt(jnp.maximum(jnp.sum(xm * xm, axis=1, keepdims=True), EPS))
    sim = lax.dot_general(xm_n, pk_n, (((1,), (1,)), ((), ())),
                          preferred_element_type=jnp.float32)
    sim_ref[...] = sim
    mx = jnp.max(sim, axis=1, keepdims=True)
    iot = lax.broadcasted_iota(jnp.int32, (B, POOL), 1)
    idx_ref[...] = jnp.min(jnp.where(sim >= mx, iot, POOL), axis=1, keepdims=True)
    rs_ref[...] = jnp.sum(mx, keepdims=True).reshape(1, 1) / B


def _sim_call(x_mean, prompt_key):
    return pl.pallas_call(
        _sim_body,
        in_specs=[pl.BlockSpec((B, EMBED), lambda: (0, 0)),
                  pl.BlockSpec((POOL, EMBED), lambda: (0, 0))],
        out_specs=[pl.BlockSpec((B, POOL), lambda: (0, 0)),
                   pl.BlockSpec((B, 1), lambda: (0, 0)),
                   pl.BlockSpec((1, 1), lambda: (0, 0))],
        out_shape=[jax.ShapeDtypeStruct((B, POOL), jnp.float32),
                   jax.ShapeDtypeStruct((B, 1), jnp.int32),
                   jax.ShapeDtypeStruct((1, 1), jnp.float32)],
    )(x_mean, prompt_key)


_IMG_ROW = CH * SZ * SZ
_GW = 16
_BPW = B // _GW


def _gather_call(idx_flat, prompt_flat):
    info = plsc.get_sparse_core_info()
    nc = info.num_cores
    mesh = plsc.VectorSubcoreMesh(core_axis_name="c", subcore_axis_name="s")

    @functools.partial(
        pl.kernel, mesh=mesh,
        out_type=jax.ShapeDtypeStruct((B, _IMG_ROW), jnp.float32),
        scratch_types=[
            pltpu.VMEM((_BPW,), jnp.int32),
            pltpu.VMEM((_BPW, _IMG_ROW), jnp.float32),
            pltpu.SemaphoreType.DMA,
        ],
    )
    def sc_gather(table_hbm, idx_hbm, out_hbm, idx_v, rows_v, sem):
        wid = lax.axis_index("s") * nc + lax.axis_index("c")

        @pl.when(wid < _GW)
        def _():
            base = wid * _BPW
            pltpu.sync_copy(idx_hbm.at[pl.ds(base, _BPW)], idx_v)
            pltpu.async_copy(table_hbm.at[idx_v], rows_v, sem).wait()
            pltpu.sync_copy(rows_v, out_hbm.at[pl.ds(base, _BPW)])

    return sc_gather(prompt_flat, idx_flat)


def _weff_body(rxs_ref, rys_ref, wr_ref, o_ref):
    rys = rys_ref[...]
    for gx in range(G):
        rxs = rxs_ref[gx * WIN : (gx + 1) * WIN, :]
        t1 = jnp.dot(rxs, wr_ref[...], preferred_element_type=jnp.float32)
        t1 = t1.reshape(WIN, CH, PATCH, EMBED).transpose(2, 1, 0, 3)
        t1 = t1.reshape(PATCH, CH * WIN * EMBED)
        t2 = jnp.dot(rys, t1, preferred_element_type=jnp.float32)
        t2 = t2.reshape(G, WIN, CH, WIN, EMBED).transpose(0, 2, 1, 3, 4)
        t2 = t2.reshape(G, K, EMBED)
        for gy in range(G):
            g = gy * G + gx
            o_ref[g // RB, g % RB] = t2[gy]


def _weff_call(rxs2, rys2, wr):
    return pl.pallas_call(
        _weff_body,
        in_specs=[pl.BlockSpec((G * WIN, PATCH), lambda: (0, 0)),
                  pl.BlockSpec((G * WIN, PATCH), lambda: (0, 0)),
                  pl.BlockSpec((PATCH, CH * PATCH * EMBED), lambda: (0, 0))],
        out_specs=pl.BlockSpec((NEB + 1, RB, K, EMBED), lambda: (0, 0, 0, 0)),
        out_shape=jax.ShapeDtypeStruct((NEB + 1, RB, K, EMBED), jnp.float32),
    )(rxs2, rys2, wr)


def _xall_body(y0_ref, imgs_ref, o_ref):
    gy = pl.program_id(0)
    y0 = y0_ref[gy]
    win_y = imgs_ref[:, :, pl.ds(y0, WIN), :]
    for gx in range(G):
        x0 = _X0[gx]
        win = win_y[:, :, :, x0 : x0 + WIN]
        o_ref[0, gx] = win.reshape(B, K)


def _xall_call(y0_tab, imgs):
    grid_spec = pltpu.PrefetchScalarGridSpec(
        num_scalar_prefetch=1,
        grid=(G,),
        in_specs=[pl.BlockSpec((B, CH, SZ, SZ), lambda gy, y0: (0, 0, 0, 0))],
        out_specs=pl.BlockSpec((1, G, B, K), lambda gy, y0: (gy, 0, 0, 0)),
    )
    return pl.pallas_call(
        _xall_body,
        grid_spec=grid_spec,
        out_shape=jax.ShapeDtypeStruct((G, G, B, K), jnp.float32),
    )(y0_tab, imgs)


def _main_body(xa_ref, xb_ref, weff_ref, b_ref, x_ref, o_in_ref, out_ref):
    del o_in_ref
    j = pl.program_id(0)
    bias = b_ref[...]

    def embed_row(r):
        xmat = xa_ref[0, r] if r < WIN else xb_ref[0, r - WIN]
        w = weff_ref[0, r]
        out_ref[:, 0, r, :] = (
            jnp.dot(xmat, w, preferred_element_type=jnp.float32) + bias)

    @pl.when(j < NEB)
    def _embed_full():
        for r in range(RB):
            embed_row(r)

    @pl.when(j == NEB)
    def _mixed():
        for r in range(RB - WIN):
            embed_row(r)
        out_ref[:, 0, RB - WIN :, :] = x_ref[:, : WIN, :]


def _main_call(xall, weff, patch_b, x_embed, out_buf):
    nxb = L // WIN
    return pl.pallas_call(
        _main_body,
        grid=(NEB + 1,),
        in_specs=[
            pl.BlockSpec((1, WIN, B, K),
                         lambda j: (jnp.minimum(2 * j, nxb - 1), 0, 0, 0)),
            pl.BlockSpec((1, WIN, B, K),
                         lambda j: (jnp.minimum(2 * j + 1, nxb - 1), 0, 0, 0)),
            pl.BlockSpec((1, RB, K, EMBED), lambda j: (j, 0, 0, 0)),
            pl.BlockSpec((1, EMBED), lambda j: (0, 0)),
            pl.BlockSpec((B, RB, EMBED), lambda j: (0, 0, 0)),
            pl.BlockSpec((1, 1, RB, EMBED), lambda j: (0, 0, 0, 0)),
        ],
        out_specs=pl.BlockSpec((B, 1, RB, EMBED), lambda j: (0, j, 0, 0)),
        out_shape=jax.ShapeDtypeStruct((B, NB, RB, EMBED), jnp.float32),
        input_output_aliases={5: 0},
    )(xall, xall, weff, patch_b.reshape(1, EMBED), x_embed, out_buf)


def kernel(x_embed, prompt, prompt_key, patch_W, patch_b):
    rxs2 = jnp.asarray(_RS.reshape(G * WIN, PATCH))
    y0_tab = jnp.asarray(_Y0)
    wr = (patch_W.reshape(CH, PATCH, PATCH, EMBED)
          .transpose(2, 0, 1, 3).reshape(PATCH, CH * PATCH * EMBED))

    x_mean, out_buf = _mean_call(x_embed)
    similarity, idx, rs = _sim_call(x_mean, prompt_key)
    imgs_flat = _gather_call(idx[:, 0], prompt.reshape(POOL, _IMG_ROW))
    imgs = imgs_flat.reshape(B, CH, SZ, SZ)
    weff = _weff_call(rxs2, rxs2, wr)
    xall = _xall_call(y0_tab, imgs).reshape(L // WIN, WIN, B, K)
    out4 = _main_call(xall, weff, patch_b, x_embed, out_buf)
    prompted = out4.reshape(B, 2 * L, EMBED)
    return (prompted, rs.reshape(()), similarity, idx,
            imgs.reshape(B, 1, CH, SZ, SZ))


# speedup vs baseline: 1.3410x; 1.3410x over previous
import functools

import numpy as np

import jax
import jax.numpy as jnp
from jax import lax
from jax.experimental import pallas as pl
from jax.experimental.pallas import tpu as pltpu
from jax.experimental.pallas import tpu_sc as plsc

POOL = 256
CH = 3
SZ = 32
EMBED = 768
B = 128
L = 196
PATCH = 16
IMG = 224
G = IMG // PATCH
WIN = 4
K = CH * WIN * WIN
EPS = 1e-12
RB = 8
NEB = L // RB
NB = 2 * L // RB


def _resize_mat():
    x = np.arange(IMG)
    s = (x + 0.5) * (SZ / IMG) - 0.5
    lo = np.floor(s).astype(int)
    w = s - lo
    r = np.zeros((IMG, SZ), np.float64)
    for i in range(IMG):
        for j, wt in ((lo[i], 1.0 - w[i]), (lo[i] + 1, w[i])):
            r[i, min(max(j, 0), SZ - 1)] += wt
    return r


_RY = _resize_mat()
_P0 = np.zeros(G, np.int32)
_RS = np.zeros((G, WIN, PATCH), np.float32)
for _g in range(G):
    _rows = _RY[PATCH * _g : PATCH * (_g + 1)]
    _nz = np.nonzero(np.abs(_rows).sum(0))[0]
    _p0 = min(int(_nz.min()), SZ - WIN)
    _P0[_g] = _p0
    _RS[_g] = _rows[:, _p0 : _p0 + WIN].T.astype(np.float32)

_Y0 = _P0.copy()
_X0 = [int(v) for v in _P0]


_MB = 8


def _mean_body(x_ref, o_ref, xs_ref):
    x = x_ref[...]
    o_ref[...] = jnp.mean(x, axis=1)
    xs_ref[:, RB - WIN :, :] = x
    xs_ref[:, : RB - WIN, :] = jnp.zeros_like(xs_ref[:, : RB - WIN, :])


def _mean_call(x_embed):
    return pl.pallas_call(
        _mean_body,
        grid=(B // _MB,),
        in_specs=[pl.BlockSpec((_MB, L, EMBED), lambda i: (i, 0, 0))],
        out_specs=[pl.BlockSpec((_MB, EMBED), lambda i: (i, 0)),
                   pl.BlockSpec((_MB, L + 4, EMBED), lambda i: (i, 0, 0))],
        out_shape=[jax.ShapeDtypeStruct((B, EMBED), jnp.float32),
                   jax.ShapeDtypeStruct((B, L + 4, EMBED), jnp.float32)],
    )(x_embed)


def _sim_body(xm_ref, pk_ref, sim_ref, idx_ref, rs_ref):
    xm = xm_ref[...]
    pk = pk_ref[...]
    pk_n = pk * lax.rsqrt(jnp.maximum(jnp.sum(pk * pk, axis=1, keepdims=True), EPS))
    xm_n = xm * lax.rsqrt(jnp.maximum(jnp.sum(xm * xm, axis=1, keepdims=True), EPS))
    sim = lax.dot_general(xm_n, pk_n, (((1,), (1,)), ((), ())),
                          preferred_element_type=jnp.float32)
    sim_ref[...] = sim
    mx = jnp.max(sim, axis=1, keepdims=True)
    iot = lax.broadcasted_iota(jnp.int32, (B, POOL), 1)
    idx_ref[...] = jnp.min(jnp.where(sim >= mx, iot, POOL), axis=1, keepdims=True)
    rs_ref[...] = jnp.sum(mx, keepdims=True).reshape(1, 1) / B


def _sim_call(x_mean, prompt_key):
    return pl.pallas_call(
        _sim_body,
        in_specs=[pl.BlockSpec((B, EMBED), lambda: (0, 0)),
                  pl.BlockSpec((POOL, EMBED), lambda: (0, 0))],
        out_specs=[pl.BlockSpec((B, POOL), lambda: (0, 0)),
                   pl.BlockSpec((B, 1), lambda: (0, 0)),
                   pl.BlockSpec((1, 1), lambda: (0, 0))],
        out_shape=[jax.ShapeDtypeStruct((B, POOL), jnp.float32),
                   jax.ShapeDtypeStruct((B, 1), jnp.int32),
                   jax.ShapeDtypeStruct((1, 1), jnp.float32)],
    )(x_mean, prompt_key)


_IMG_ROW = CH * SZ * SZ
_GW = 16
_BPW = B // _GW


def _gather_call(idx_flat, prompt_flat):
    info = plsc.get_sparse_core_info()
    nc = info.num_cores
    mesh = plsc.VectorSubcoreMesh(core_axis_name="c", subcore_axis_name="s")

    @functools.partial(
        pl.kernel, mesh=mesh,
        out_type=jax.ShapeDtypeStruct((B, _IMG_ROW), jnp.float32),
        scratch_types=[
            pltpu.VMEM((_BPW,), jnp.int32),
            pltpu.VMEM((_BPW, _IMG_ROW), jnp.float32),
            pltpu.SemaphoreType.DMA,
        ],
    )
    def sc_gather(table_hbm, idx_hbm, out_hbm, idx_v, rows_v, sem):
        wid = lax.axis_index("s") * nc + lax.axis_index("c")

        @pl.when(wid < _GW)
        def _():
            base = wid * _BPW
            pltpu.sync_copy(idx_hbm.at[pl.ds(base, _BPW)], idx_v)
            pltpu.async_copy(table_hbm.at[idx_v], rows_v, sem).wait()
            pltpu.sync_copy(rows_v, out_hbm.at[pl.ds(base, _BPW)])

    return sc_gather(prompt_flat, idx_flat)


def _main_body(rxs_ref, wr_ref, imgs_ref, b_ref, xs_ref, out_ref,
               weff_s, xall_s):
    j = pl.program_id(0)
    bias = b_ref[...]

    @pl.when(j == 0)
    def _prep():
        rys = rxs_ref[...]
        for gx in range(G):
            rxs = rxs_ref[gx * WIN : (gx + 1) * WIN, :]
            t1 = jnp.dot(rxs, wr_ref[...], preferred_element_type=jnp.float32)
            t1 = t1.reshape(WIN, CH, PATCH, EMBED).transpose(2, 1, 0, 3)
            t1 = t1.reshape(PATCH, CH * WIN * EMBED)
            t2 = jnp.dot(rys, t1, preferred_element_type=jnp.float32)
            t2 = t2.reshape(G, WIN, CH, WIN, EMBED).transpose(0, 2, 1, 3, 4)
            t2 = t2.reshape(G, K, EMBED)
            for gy in range(G):
                g = gy * G + gx
                weff_s[g // RB, g % RB] = t2[gy].astype(jnp.bfloat16)
        for gy in range(G):
            y0 = int(_Y0[gy])
            win_y = imgs_ref[:, :, y0 : y0 + WIN, :]
            for gx in range(G):
                x0 = _X0[gx]
                win = win_y[:, :, :, x0 : x0 + WIN]
                g = gy * G + gx
                xall_s[g // WIN, g % WIN] = win.reshape(B, K).astype(jnp.bfloat16)

    def embed_row(r):
        xmat = xall_s[2 * j + r // WIN, r % WIN]
        w = weff_s[j, r]
        out_ref[:, 0, r, :] = (
            jnp.dot(xmat, w, preferred_element_type=jnp.float32) + bias)

    @pl.when(j < NEB)
    def _embed_full():
        for r in range(RB):
            embed_row(r)

    @pl.when(j == NEB)
    def _mixed():
        for r in range(RB - WIN):
            embed_row(r)
        out_ref[:, 0, RB - WIN :, :] = xs_ref[:, RB - WIN :, :]

    @pl.when(j > NEB)
    def _copy():
        out_ref[:, 0, :, :] = xs_ref[...]


def _main_call(rxs2, wr, imgs, patch_b, x_shift):
    return pl.pallas_call(
        _main_body,
        grid=(NB,),
        in_specs=[
            pl.BlockSpec((G * WIN, PATCH), lambda j: (0, 0)),
            pl.BlockSpec((PATCH, CH * PATCH * EMBED), lambda j: (0, 0)),
            pl.BlockSpec((B, CH, SZ, SZ), lambda j: (0, 0, 0, 0)),
            pl.BlockSpec((1, EMBED), lambda j: (0, 0)),
            pl.BlockSpec((B, RB, EMBED),
                         lambda j: (0, jnp.maximum(j - NEB, 0), 0)),
        ],
        out_specs=pl.BlockSpec((B, 1, RB, EMBED), lambda j: (0, j, 0, 0)),
        out_shape=jax.ShapeDtypeStruct((B, NB, RB, EMBED), jnp.float32),
        scratch_shapes=[
            pltpu.VMEM((NEB + 1, RB, K, EMBED), jnp.bfloat16),
            pltpu.VMEM((L // WIN, WIN, B, K), jnp.bfloat16),
        ],
    )(rxs2, wr, imgs, patch_b.reshape(1, EMBED), x_shift)


def kernel(x_embed, prompt, prompt_key, patch_W, patch_b):
    rxs2 = jnp.asarray(_RS.reshape(G * WIN, PATCH))
    wr = (patch_W.reshape(CH, PATCH, PATCH, EMBED)
          .transpose(2, 0, 1, 3).reshape(PATCH, CH * PATCH * EMBED))

    x_mean, x_shift = _mean_call(x_embed)
    similarity, idx, rs = _sim_call(x_mean, prompt_key)
    imgs_flat = _gather_call(idx[:, 0], prompt.reshape(POOL, _IMG_ROW))
    imgs = imgs_flat.reshape(B, CH, SZ, SZ)
    out4 = _main_call(rxs2, wr, imgs, patch_b, x_shift)
    prompted = out4.reshape(B, 2 * L, EMBED)
    return (prompted, rs.reshape(()), similarity, idx,
            imgs.reshape(B, 1, CH, SZ, SZ))


# speedup vs baseline: 1.3543x; 1.0099x over previous
import functools

import numpy as np

import jax
import jax.numpy as jnp
from jax import lax
from jax.experimental import pallas as pl
from jax.experimental.pallas import tpu as pltpu
from jax.experimental.pallas import tpu_sc as plsc

POOL = 256
CH = 3
SZ = 32
EMBED = 768
B = 128
L = 196
PATCH = 16
IMG = 224
G = IMG // PATCH
WIN = 4
K = CH * WIN * WIN
EPS = 1e-12
RB = 8
NEB = L // RB
NB = 2 * L // RB


def _resize_mat():
    x = np.arange(IMG)
    s = (x + 0.5) * (SZ / IMG) - 0.5
    lo = np.floor(s).astype(int)
    w = s - lo
    r = np.zeros((IMG, SZ), np.float64)
    for i in range(IMG):
        for j, wt in ((lo[i], 1.0 - w[i]), (lo[i] + 1, w[i])):
            r[i, min(max(j, 0), SZ - 1)] += wt
    return r


_RY = _resize_mat()
_P0 = np.zeros(G, np.int32)
_RS = np.zeros((G, WIN, PATCH), np.float32)
for _g in range(G):
    _rows = _RY[PATCH * _g : PATCH * (_g + 1)]
    _nz = np.nonzero(np.abs(_rows).sum(0))[0]
    _p0 = min(int(_nz.min()), SZ - WIN)
    _P0[_g] = _p0
    _RS[_g] = _rows[:, _p0 : _p0 + WIN].T.astype(np.float32)

_Y0 = _P0.copy()
_X0 = [int(v) for v in _P0]


_MB = 8


def _mean_body(x_ref, o_ref):
    o_ref[...] = jnp.mean(x_ref[...], axis=1)


def _mean_call(x_embed):
    return pl.pallas_call(
        _mean_body,
        grid=(B // _MB,),
        in_specs=[pl.BlockSpec((_MB, L, EMBED), lambda i: (i, 0, 0))],
        out_specs=pl.BlockSpec((_MB, EMBED), lambda i: (i, 0)),
        out_shape=jax.ShapeDtypeStruct((B, EMBED), jnp.float32),
    )(x_embed)


def _sim_body(xm_ref, pk_ref, sim_ref, idx_ref, rs_ref):
    xm = xm_ref[...]
    pk = pk_ref[...]
    pk_n = pk * lax.rsqrt(jnp.maximum(jnp.sum(pk * pk, axis=1, keepdims=True), EPS))
    xm_n = xm * lax.rsqrt(jnp.maximum(jnp.sum(xm * xm, axis=1, keepdims=True), EPS))
    sim = lax.dot_general(xm_n, pk_n, (((1,), (1,)), ((), ())),
                          preferred_element_type=jnp.float32)
    sim_ref[...] = sim
    mx = jnp.max(sim, axis=1, keepdims=True)
    iot = lax.broadcasted_iota(jnp.int32, (B, POOL), 1)
    idx_ref[...] = jnp.min(jnp.where(sim >= mx, iot, POOL), axis=1, keepdims=True)
    rs_ref[...] = jnp.sum(mx, keepdims=True).reshape(1, 1) / B


def _sim_call(x_mean, prompt_key):
    return pl.pallas_call(
        _sim_body,
        in_specs=[pl.BlockSpec((B, EMBED), lambda: (0, 0)),
                  pl.BlockSpec((POOL, EMBED), lambda: (0, 0))],
        out_specs=[pl.BlockSpec((B, POOL), lambda: (0, 0)),
                   pl.BlockSpec((B, 1), lambda: (0, 0)),
                   pl.BlockSpec((1, 1), lambda: (0, 0))],
        out_shape=[jax.ShapeDtypeStruct((B, POOL), jnp.float32),
                   jax.ShapeDtypeStruct((B, 1), jnp.int32),
                   jax.ShapeDtypeStruct((1, 1), jnp.float32)],
    )(x_mean, prompt_key)


_IMG_ROW = CH * SZ * SZ
_GW = 16
_BPW = B // _GW


def _gather_call(idx_flat, prompt_flat):
    info = plsc.get_sparse_core_info()
    nc = info.num_cores
    mesh = plsc.VectorSubcoreMesh(core_axis_name="c", subcore_axis_name="s")

    @functools.partial(
        pl.kernel, mesh=mesh,
        out_type=jax.ShapeDtypeStruct((B, _IMG_ROW), jnp.float32),
        scratch_types=[
            pltpu.VMEM((_BPW,), jnp.int32),
            pltpu.VMEM((_BPW, _IMG_ROW), jnp.float32),
            pltpu.SemaphoreType.DMA,
        ],
    )
    def sc_gather(table_hbm, idx_hbm, out_hbm, idx_v, rows_v, sem):
        wid = lax.axis_index("s") * nc + lax.axis_index("c")

        @pl.when(wid < _GW)
        def _():
            base = wid * _BPW
            pltpu.sync_copy(idx_hbm.at[pl.ds(base, _BPW)], idx_v)
            pltpu.async_copy(table_hbm.at[idx_v], rows_v, sem).wait()
            pltpu.sync_copy(rows_v, out_hbm.at[pl.ds(base, _BPW)])

    return sc_gather(prompt_flat, idx_flat)


def _main_body(rxs_ref, wr_ref, imgs_ref, b_ref, xa_ref, xb_ref, out_ref,
               weff_s, xall_s):
    j = pl.program_id(0)
    bias = b_ref[...]

    @pl.when(j == 0)
    def _prep():
        rys = rxs_ref[...]
        for gx in range(G):
            rxs = rxs_ref[gx * WIN : (gx + 1) * WIN, :]
            t1 = jnp.dot(rxs, wr_ref[...], preferred_element_type=jnp.float32)
            t1 = t1.reshape(WIN, CH, PATCH, EMBED).transpose(2, 1, 0, 3)
            t1 = t1.reshape(PATCH, CH * WIN * EMBED)
            t2 = jnp.dot(rys, t1, preferred_element_type=jnp.float32)
            t2 = t2.reshape(G, WIN, CH, WIN, EMBED).transpose(0, 2, 1, 3, 4)
            t2 = t2.reshape(G, K, EMBED)
            for gy in range(G):
                g = gy * G + gx
                weff_s[g // RB, g % RB] = t2[gy].astype(jnp.bfloat16)
        for gy in range(G):
            y0 = int(_Y0[gy])
            win_y = imgs_ref[:, :, y0 : y0 + WIN, :]
            for gx in range(G):
                x0 = _X0[gx]
                win = win_y[:, :, :, x0 : x0 + WIN]
                g = gy * G + gx
                xall_s[g // WIN, g % WIN] = win.reshape(B, K).astype(jnp.bfloat16)

    def embed_row(r):
        xmat = xall_s[2 * j + r // WIN, r % WIN]
        w = weff_s[j, r]
        out_ref[:, 0, r, :] = (
            jnp.dot(xmat, w, preferred_element_type=jnp.float32) + bias)

    @pl.when(j < NEB)
    def _embed_full():
        for r in range(RB):
            embed_row(r)

    @pl.when(j == NEB)
    def _mixed():
        for r in range(RB - WIN):
            embed_row(r)
        out_ref[:, 0, RB - WIN :, :] = xb_ref[:, : WIN, :]

    @pl.when(j > NEB)
    def _copy():
        out_ref[:, 0, :, :] = jnp.concatenate(
            [xa_ref[:, WIN:, :], xb_ref[:, : WIN, :]], axis=1)


def _main_call(rxs2, wr, imgs, patch_b, x_embed):
    return pl.pallas_call(
        _main_body,
        grid=(NB,),
        in_specs=[
            pl.BlockSpec((G * WIN, PATCH), lambda j: (0, 0)),
            pl.BlockSpec((PATCH, CH * PATCH * EMBED), lambda j: (0, 0)),
            pl.BlockSpec((B, CH, SZ, SZ), lambda j: (0, 0, 0, 0)),
            pl.BlockSpec((1, EMBED), lambda j: (0, 0)),
            pl.BlockSpec((B, RB, EMBED),
                         lambda j: (0, jnp.maximum(j - NEB - 1, 0), 0)),
            pl.BlockSpec((B, RB, EMBED),
                         lambda j: (0, jnp.clip(j - NEB, 0, NEB), 0)),
        ],
        out_specs=pl.BlockSpec((B, 1, RB, EMBED), lambda j: (0, j, 0, 0)),
        out_shape=jax.ShapeDtypeStruct((B, NB, RB, EMBED), jnp.float32),
        scratch_shapes=[
            pltpu.VMEM((NEB + 1, RB, K, EMBED), jnp.bfloat16),
            pltpu.VMEM((L // WIN, WIN, B, K), jnp.bfloat16),
        ],
    )(rxs2, wr, imgs, patch_b.reshape(1, EMBED), x_embed, x_embed)


def kernel(x_embed, prompt, prompt_key, patch_W, patch_b):
    rxs2 = jnp.asarray(_RS.reshape(G * WIN, PATCH))
    wr = (patch_W.reshape(CH, PATCH, PATCH, EMBED)
          .transpose(2, 0, 1, 3).reshape(PATCH, CH * PATCH * EMBED))

    x_mean = _mean_call(x_embed)
    similarity, idx, rs = _sim_call(x_mean, prompt_key)
    imgs_flat = _gather_call(idx[:, 0], prompt.reshape(POOL, _IMG_ROW))
    imgs = imgs_flat.reshape(B, CH, SZ, SZ)
    out4 = _main_call(rxs2, wr, imgs, patch_b, x_embed)
    prompted = out4.reshape(B, 2 * L, EMBED)
    return (prompted, rs.reshape(()), similarity, idx,
            imgs.reshape(B, 1, CH, SZ, SZ))


# speedup vs baseline: 1.3619x; 1.0056x over previous
import functools

import numpy as np

import jax
import jax.numpy as jnp
from jax import lax
from jax.experimental import pallas as pl
from jax.experimental.pallas import tpu as pltpu
from jax.experimental.pallas import tpu_sc as plsc

POOL = 256
CH = 3
SZ = 32
EMBED = 768
B = 128
L = 196
PATCH = 16
IMG = 224
G = IMG // PATCH
WIN = 4
K = CH * WIN * WIN
EPS = 1e-12
RB = 8
NEB = L // RB
NB = 2 * L // RB


def _resize_mat():
    x = np.arange(IMG)
    s = (x + 0.5) * (SZ / IMG) - 0.5
    lo = np.floor(s).astype(int)
    w = s - lo
    r = np.zeros((IMG, SZ), np.float64)
    for i in range(IMG):
        for j, wt in ((lo[i], 1.0 - w[i]), (lo[i] + 1, w[i])):
            r[i, min(max(j, 0), SZ - 1)] += wt
    return r


_RY = _resize_mat()
_P0 = np.zeros(G, np.int32)
_RS = np.zeros((G, WIN, PATCH), np.float32)
for _g in range(G):
    _rows = _RY[PATCH * _g : PATCH * (_g + 1)]
    _nz = np.nonzero(np.abs(_rows).sum(0))[0]
    _p0 = min(int(_nz.min()), SZ - WIN)
    _P0[_g] = _p0
    _RS[_g] = _rows[:, _p0 : _p0 + WIN].T.astype(np.float32)

_Y0 = _P0.copy()
_X0 = [int(v) for v in _P0]


_MB = 8


def _sim_body(x_ref, pk_ref, sim_ref, idx_ref, rs_ref, xm_s):
    i = pl.program_id(0)

    @pl.when(i < B // _MB)
    def _mean():
        xm_s[pl.ds(i * _MB, _MB), :] = jnp.mean(x_ref[...], axis=1)

    @pl.when(i == B // _MB)
    def _sim():
        xm = xm_s[...]
        pk = pk_ref[...]
        pk_n = pk * lax.rsqrt(
            jnp.maximum(jnp.sum(pk * pk, axis=1, keepdims=True), EPS))
        xm_n = xm * lax.rsqrt(
            jnp.maximum(jnp.sum(xm * xm, axis=1, keepdims=True), EPS))
        sim = lax.dot_general(xm_n, pk_n, (((1,), (1,)), ((), ())),
                              preferred_element_type=jnp.float32)
        sim_ref[...] = sim
        mx = jnp.max(sim, axis=1, keepdims=True)
        iot = lax.broadcasted_iota(jnp.int32, (B, POOL), 1)
        idx_ref[...] = jnp.min(jnp.where(sim >= mx, iot, POOL), axis=1,
                               keepdims=True)
        rs_ref[...] = jnp.sum(mx, keepdims=True).reshape(1, 1) / B


def _sim_call(x_embed, prompt_key):
    nmb = B // _MB
    return pl.pallas_call(
        _sim_body,
        grid=(nmb + 1,),
        in_specs=[
            pl.BlockSpec((_MB, L, EMBED),
                         lambda i: (jnp.minimum(i, nmb - 1), 0, 0)),
            pl.BlockSpec((POOL, EMBED), lambda i: (0, 0)),
        ],
        out_specs=[pl.BlockSpec((B, POOL), lambda i: (0, 0)),
                   pl.BlockSpec((B, 1), lambda i: (0, 0)),
                   pl.BlockSpec((1, 1), lambda i: (0, 0))],
        out_shape=[jax.ShapeDtypeStruct((B, POOL), jnp.float32),
                   jax.ShapeDtypeStruct((B, 1), jnp.int32),
                   jax.ShapeDtypeStruct((1, 1), jnp.float32)],
        scratch_shapes=[pltpu.VMEM((B, EMBED), jnp.float32)],
    )(x_embed, prompt_key)


_IMG_ROW = CH * SZ * SZ
_GW = 16
_BPW = B // _GW


def _gather_call(idx_flat, prompt_flat):
    info = plsc.get_sparse_core_info()
    nc = info.num_cores
    mesh = plsc.VectorSubcoreMesh(core_axis_name="c", subcore_axis_name="s")

    @functools.partial(
        pl.kernel, mesh=mesh,
        out_type=jax.ShapeDtypeStruct((B, _IMG_ROW), jnp.float32),
        scratch_types=[
            pltpu.VMEM((_BPW,), jnp.int32),
            pltpu.VMEM((_BPW, _IMG_ROW), jnp.float32),
            pltpu.SemaphoreType.DMA,
        ],
    )
    def sc_gather(table_hbm, idx_hbm, out_hbm, idx_v, rows_v, sem):
        wid = lax.axis_index("s") * nc + lax.axis_index("c")

        @pl.when(wid < _GW)
        def _():
            base = wid * _BPW
            pltpu.sync_copy(idx_hbm.at[pl.ds(base, _BPW)], idx_v)
            pltpu.async_copy(table_hbm.at[idx_v], rows_v, sem).wait()
            pltpu.sync_copy(rows_v, out_hbm.at[pl.ds(base, _BPW)])

    return sc_gather(prompt_flat, idx_flat)


def _main_body(rxs_ref, wr_ref, imgs_ref, b_ref, xa_ref, xb_ref, out_ref,
               weff_s, xall_s):
    j = pl.program_id(0)
    bias = b_ref[...]

    @pl.when(j == 0)
    def _prep():
        rys = rxs_ref[...]
        for gx in range(G):
            rxs = rxs_ref[gx * WIN : (gx + 1) * WIN, :]
            t1 = jnp.dot(rxs, wr_ref[...], preferred_element_type=jnp.float32)
            t1 = t1.reshape(WIN, CH, PATCH, EMBED).transpose(2, 1, 0, 3)
            t1 = t1.reshape(PATCH, CH * WIN * EMBED)
            t2 = jnp.dot(rys, t1, preferred_element_type=jnp.float32)
            t2 = t2.reshape(G, WIN, CH, WIN, EMBED).transpose(0, 2, 1, 3, 4)
            t2 = t2.reshape(G, K, EMBED)
            for gy in range(G):
                g = gy * G + gx
                weff_s[g // RB, g % RB] = t2[gy].astype(jnp.bfloat16)
        for gy in range(G):
            y0 = int(_Y0[gy])
            win_y = imgs_ref[:, :, y0 : y0 + WIN, :]
            for gx in range(G):
                x0 = _X0[gx]
                win = win_y[:, :, :, x0 : x0 + WIN]
                g = gy * G + gx
                xall_s[g // WIN, g % WIN] = win.reshape(B, K).astype(jnp.bfloat16)

    def embed_row(r):
        xmat = xall_s[2 * j + r // WIN, r % WIN]
        w = weff_s[j, r]
        out_ref[:, 0, r, :] = (
            jnp.dot(xmat, w, preferred_element_type=jnp.float32) + bias)

    @pl.when(j < NEB)
    def _embed_full():
        for r in range(RB):
            embed_row(r)

    @pl.when(j == NEB)
    def _mixed():
        for r in range(RB - WIN):
            embed_row(r)
        out_ref[:, 0, RB - WIN :, :] = xb_ref[:, : WIN, :]

    @pl.when(j > NEB)
    def _copy():
        out_ref[:, 0, :, :] = jnp.concatenate(
            [xa_ref[:, WIN:, :], xb_ref[:, : WIN, :]], axis=1)


def _main_call(rxs2, wr, imgs, patch_b, x_embed):
    return pl.pallas_call(
        _main_body,
        grid=(NB,),
        in_specs=[
            pl.BlockSpec((G * WIN, PATCH), lambda j: (0, 0)),
            pl.BlockSpec((PATCH, CH * PATCH * EMBED), lambda j: (0, 0)),
            pl.BlockSpec((B, CH, SZ, SZ), lambda j: (0, 0, 0, 0)),
            pl.BlockSpec((1, EMBED), lambda j: (0, 0)),
            pl.BlockSpec((B, RB, EMBED),
                         lambda j: (0, jnp.maximum(j - NEB - 1, 0), 0)),
            pl.BlockSpec((B, RB, EMBED),
                         lambda j: (0, jnp.clip(j - NEB, 0, NEB), 0)),
        ],
        out_specs=pl.BlockSpec((B, 1, RB, EMBED), lambda j: (0, j, 0, 0)),
        out_shape=jax.ShapeDtypeStruct((B, NB, RB, EMBED), jnp.float32),
        scratch_shapes=[
            pltpu.VMEM((NEB + 1, RB, K, EMBED), jnp.bfloat16),
            pltpu.VMEM((L // WIN, WIN, B, K), jnp.bfloat16),
        ],
    )(rxs2, wr, imgs, patch_b.reshape(1, EMBED), x_embed, x_embed)


def kernel(x_embed, prompt, prompt_key, patch_W, patch_b):
    rxs2 = jnp.asarray(_RS.reshape(G * WIN, PATCH))
    wr = (patch_W.reshape(CH, PATCH, PATCH, EMBED)
          .transpose(2, 0, 1, 3).reshape(PATCH, CH * PATCH * EMBED))

    similarity, idx, rs = _sim_call(x_embed, prompt_key)
    imgs_flat = _gather_call(idx[:, 0], prompt.reshape(POOL, _IMG_ROW))
    imgs = imgs_flat.reshape(B, CH, SZ, SZ)
    out4 = _main_call(rxs2, wr, imgs, patch_b, x_embed)
    prompted = out4.reshape(B, 2 * L, EMBED)
    return (prompted, rs.reshape(()), similarity, idx,
            imgs.reshape(B, 1, CH, SZ, SZ))


# speedup vs baseline: 1.4676x; 1.0776x over previous
import functools

import numpy as np

import jax
import jax.numpy as jnp
from jax import lax
from jax.experimental import pallas as pl
from jax.experimental.pallas import tpu as pltpu
from jax.experimental.pallas import tpu_sc as plsc

POOL = 256
CH = 3
SZ = 32
EMBED = 768
B = 128
L = 196
PATCH = 16
IMG = 224
G = IMG // PATCH
WIN = 4
K = CH * WIN * WIN
EPS = 1e-12
RB = 8
NEB = L // RB
NB = 2 * L // RB


def _resize_mat():
    x = np.arange(IMG)
    s = (x + 0.5) * (SZ / IMG) - 0.5
    lo = np.floor(s).astype(int)
    w = s - lo
    r = np.zeros((IMG, SZ), np.float64)
    for i in range(IMG):
        for j, wt in ((lo[i], 1.0 - w[i]), (lo[i] + 1, w[i])):
            r[i, min(max(j, 0), SZ - 1)] += wt
    return r


_RY = _resize_mat()
_P0 = np.zeros(G, np.int32)
_RS = np.zeros((G, WIN, PATCH), np.float32)
for _g in range(G):
    _rows = _RY[PATCH * _g : PATCH * (_g + 1)]
    _nz = np.nonzero(np.abs(_rows).sum(0))[0]
    _p0 = min(int(_nz.min()), SZ - WIN)
    _P0[_g] = _p0
    _RS[_g] = _rows[:, _p0 : _p0 + WIN].T.astype(np.float32)

_Y0 = _P0.copy()
_X0 = [int(v) for v in _P0]


_MB = 8


def _sim_body(x_ref, pk_ref, sim_ref, idx_ref, rs_ref, xm_s):
    i = pl.program_id(0)

    @pl.when(i < B // _MB)
    def _mean():
        xm_s[pl.ds(i * _MB, _MB), :] = jnp.mean(x_ref[...], axis=1)

    @pl.when(i == B // _MB)
    def _sim():
        xm = xm_s[...]
        pk = pk_ref[...]
        pk_n = pk * lax.rsqrt(
            jnp.maximum(jnp.sum(pk * pk, axis=1, keepdims=True), EPS))
        xm_n = xm * lax.rsqrt(
            jnp.maximum(jnp.sum(xm * xm, axis=1, keepdims=True), EPS))
        sim = lax.dot_general(xm_n, pk_n, (((1,), (1,)), ((), ())),
                              preferred_element_type=jnp.float32)
        sim_ref[...] = sim
        mx = jnp.max(sim, axis=1, keepdims=True)
        iot = lax.broadcasted_iota(jnp.int32, (B, POOL), 1)
        idx_ref[...] = jnp.min(jnp.where(sim >= mx, iot, POOL), axis=1,
                               keepdims=True)
        rs_ref[...] = jnp.sum(mx, keepdims=True).reshape(1, 1) / B


def _sim_call(x_embed, prompt_key):
    nmb = B // _MB
    return pl.pallas_call(
        _sim_body,
        grid=(nmb + 1,),
        in_specs=[
            pl.BlockSpec((_MB, L, EMBED),
                         lambda i: (jnp.minimum(i, nmb - 1), 0, 0)),
            pl.BlockSpec((POOL, EMBED), lambda i: (0, 0)),
        ],
        out_specs=[pl.BlockSpec((B, POOL), lambda i: (0, 0)),
                   pl.BlockSpec((B, 1), lambda i: (0, 0)),
                   pl.BlockSpec((1, 1), lambda i: (0, 0))],
        out_shape=[jax.ShapeDtypeStruct((B, POOL), jnp.float32),
                   jax.ShapeDtypeStruct((B, 1), jnp.int32),
                   jax.ShapeDtypeStruct((1, 1), jnp.float32)],
        scratch_shapes=[pltpu.VMEM((B, EMBED), jnp.float32)],
    )(x_embed, prompt_key)


_IMG_ROW = CH * SZ * SZ
_GW = 16
_BPW = B // _GW


def _gather_call(idx_flat, prompt_flat):
    info = plsc.get_sparse_core_info()
    nc = info.num_cores
    mesh = plsc.VectorSubcoreMesh(core_axis_name="c", subcore_axis_name="s")

    @functools.partial(
        pl.kernel, mesh=mesh,
        out_type=jax.ShapeDtypeStruct((B, _IMG_ROW), jnp.float32),
        scratch_types=[
            pltpu.VMEM((_BPW,), jnp.int32),
            pltpu.VMEM((_BPW, _IMG_ROW), jnp.float32),
            pltpu.SemaphoreType.DMA,
        ],
    )
    def sc_gather(table_hbm, idx_hbm, out_hbm, idx_v, rows_v, sem):
        wid = lax.axis_index("s") * nc + lax.axis_index("c")

        @pl.when(wid < _GW)
        def _():
            base = wid * _BPW
            pltpu.sync_copy(idx_hbm.at[pl.ds(base, _BPW)], idx_v)
            pltpu.async_copy(table_hbm.at[idx_v], rows_v, sem).wait()
            pltpu.sync_copy(rows_v, out_hbm.at[pl.ds(base, _BPW)])

    return sc_gather(prompt_flat, idx_flat)


def _main_body(rxs_ref, wr_ref, imgs_ref, b_ref, xb_ref, out_ref,
               weff_s, xall_s, carry_s):
    j = pl.program_id(0)
    bias = b_ref[...]

    @pl.when(j == 0)
    def _prep():
        rys = rxs_ref[...]
        for gx in range(G):
            rxs = rxs_ref[gx * WIN : (gx + 1) * WIN, :]
            t1 = jnp.dot(rxs, wr_ref[...], preferred_element_type=jnp.float32)
            t1 = t1.reshape(WIN, CH, PATCH, EMBED).transpose(2, 1, 0, 3)
            t1 = t1.reshape(PATCH, CH * WIN * EMBED)
            t2 = jnp.dot(rys, t1, preferred_element_type=jnp.float32)
            t2 = t2.reshape(G, WIN, CH, WIN, EMBED).transpose(0, 2, 1, 3, 4)
            t2 = t2.reshape(G, K, EMBED)
            for gy in range(G):
                g = gy * G + gx
                weff_s[g // RB, g % RB] = t2[gy].astype(jnp.bfloat16)
        for gy in range(G):
            y0 = int(_Y0[gy])
            win_y = imgs_ref[:, :, y0 : y0 + WIN, :]
            for gx in range(G):
                x0 = _X0[gx]
                win = win_y[:, :, :, x0 : x0 + WIN]
                g = gy * G + gx
                xall_s[g // WIN, g % WIN] = win.reshape(B, K).astype(jnp.bfloat16)

    def embed_row(r):
        xmat = xall_s[2 * j + r // WIN, r % WIN]
        w = weff_s[j, r]
        out_ref[:, 0, r, :] = (
            jnp.dot(xmat, w, preferred_element_type=jnp.float32) + bias)

    @pl.when(j < NEB)
    def _embed_full():
        for r in range(RB):
            embed_row(r)

    @pl.when(j == NEB)
    def _mixed():
        for r in range(RB - WIN):
            embed_row(r)
        out_ref[:, 0, RB - WIN :, :] = xb_ref[:, : WIN, :]
        carry_s[...] = xb_ref[:, WIN:, :]

    @pl.when(j > NEB)
    def _copy():
        out_ref[:, 0, :, :] = jnp.concatenate(
            [carry_s[...], xb_ref[:, : WIN, :]], axis=1)
        carry_s[...] = xb_ref[:, WIN:, :]


def _main_call(rxs2, wr, imgs, patch_b, x_embed):
    return pl.pallas_call(
        _main_body,
        grid=(NB,),
        in_specs=[
            pl.BlockSpec((G * WIN, PATCH), lambda j: (0, 0)),
            pl.BlockSpec((PATCH, CH * PATCH * EMBED), lambda j: (0, 0)),
            pl.BlockSpec((B, CH, SZ, SZ), lambda j: (0, 0, 0, 0)),
            pl.BlockSpec((1, EMBED), lambda j: (0, 0)),
            pl.BlockSpec((B, RB, EMBED),
                         lambda j: (0, jnp.clip(j - NEB, 0, NEB), 0)),
        ],
        out_specs=pl.BlockSpec((B, 1, RB, EMBED), lambda j: (0, j, 0, 0)),
        out_shape=jax.ShapeDtypeStruct((B, NB, RB, EMBED), jnp.float32),
        scratch_shapes=[
            pltpu.VMEM((NEB + 1, RB, K, EMBED), jnp.bfloat16),
            pltpu.VMEM((L // WIN, WIN, B, K), jnp.bfloat16),
            pltpu.VMEM((B, WIN, EMBED), jnp.float32),
        ],
    )(rxs2, wr, imgs, patch_b.reshape(1, EMBED), x_embed)


def kernel(x_embed, prompt, prompt_key, patch_W, patch_b):
    rxs2 = jnp.asarray(_RS.reshape(G * WIN, PATCH))
    wr = (patch_W.reshape(CH, PATCH, PATCH, EMBED)
          .transpose(2, 0, 1, 3).reshape(PATCH, CH * PATCH * EMBED))

    similarity, idx, rs = _sim_call(x_embed, prompt_key)
    imgs_flat = _gather_call(idx[:, 0], prompt.reshape(POOL, _IMG_ROW))
    imgs = imgs_flat.reshape(B, CH, SZ, SZ)
    out4 = _main_call(rxs2, wr, imgs, patch_b, x_embed)
    prompted = out4.reshape(B, 2 * L, EMBED)
    return (prompted, rs.reshape(()), similarity, idx,
            imgs.reshape(B, 1, CH, SZ, SZ))
